# baseline (speedup 1.0000x reference)
import functools

import jax
import jax.numpy as jnp
from jax import lax
from jax.experimental import pallas as pl
from jax.experimental.pallas import tpu as pltpu
from jax.experimental.pallas import tpu_sc as plsc

DIM = 64
GAMMA = 12.0
BLK = 128
LANES = 16
NC = 2
NS = 16
NW = NC * NS

N_CLAUSES = 16384
N_SINGLES = 8192
N_CONJ = 8192


def _scores_kernel(ent, rel, ch, ct, cr, ph, pt, pr,
                   c1h, c1t, c1r, c2h, c2t, c2r,
                   out_c, out_p, out_j,
                   hidx, tidx, ridx, hrows, trows, rrows, sbuf, sbuf2, sem):
    wid = lax.axis_index("s") * NC + lax.axis_index("c")

    def gather_block(h_hbm, t_hbm, r_hbm, base):
        pltpu.sync_copy(h_hbm.at[pl.ds(base, BLK)], hidx)
        pltpu.sync_copy(t_hbm.at[pl.ds(base, BLK)], tidx)
        pltpu.sync_copy(r_hbm.at[pl.ds(base, BLK)], ridx)
        a = pltpu.async_copy(ent.at[hidx], hrows, sem)
        b = pltpu.async_copy(ent.at[tidx], trows, sem)
        c = pltpu.async_copy(rel.at[ridx], rrows, sem)
        a.wait()
        b.wait()
        c.wait()

    lane = lax.broadcasted_iota(jnp.int32, (LANES,), 0)

    def compute(out_buf):
        for g in range(BLK // LANES):

            def a_body(j, svec):
                a = g * LANES + j
                acc = jnp.zeros((LANES,), jnp.float32)
                for k in range(DIM // LANES):
                    sl = pl.ds(k * LANES, LANES)
                    acc = acc + jnp.abs(hrows[a, sl] + rrows[a, sl]
                                        - trows[a, sl])
                s = GAMMA - jnp.sum(acc)
                return jnp.where(lane == j, s, svec)

            svec = lax.fori_loop(0, LANES, a_body,
                                 jnp.zeros((LANES,), jnp.float32))
            out_buf[pl.ds(g * LANES, LANES)] = svec

    npw = N_CLAUSES // NW
    for b in range(npw // BLK):
        base = wid * npw + b * BLK
        gather_block(ch, ct, cr, base)
        compute(sbuf)
        pltpu.sync_copy(sbuf, out_c.at[pl.ds(base, BLK)])

    npw = N_SINGLES // NW
    for b in range(npw // BLK):
        base = wid * npw + b * BLK
        gather_block(ph, pt, pr, base)
        compute(sbuf)
        pltpu.sync_copy(sbuf, out_p.at[pl.ds(base, BLK)])

    npw = N_CONJ // NW
    for b in range(npw // BLK):
        base = wid * npw + b * BLK
        gather_block(c1h, c1t, c1r, base)
        compute(sbuf)
        gather_block(c2h, c2t, c2r, base)
        compute(sbuf2)
        for v in range(BLK // LANES):
            sl = pl.ds(v * LANES, LANES)
            sbuf[sl] = jnp.minimum(sbuf[sl], sbuf2[sl])
        pltpu.sync_copy(sbuf, out_j.at[pl.ds(base, BLK)])


@functools.cache
def _build():
    mesh = plsc.VectorSubcoreMesh(core_axis_name="c", subcore_axis_name="s")
    return pl.kernel(
        _scores_kernel,
        mesh=mesh,
        compiler_params=pltpu.CompilerParams(
            needs_layout_passes=False, use_tc_tiling_on_sc=False),
        out_type=[
            jax.ShapeDtypeStruct((N_CLAUSES,), jnp.float32),
            jax.ShapeDtypeStruct((N_SINGLES,), jnp.float32),
            jax.ShapeDtypeStruct((N_CONJ,), jnp.float32),
        ],
        scratch_types=[
            pltpu.VMEM((BLK,), jnp.int32),
            pltpu.VMEM((BLK,), jnp.int32),
            pltpu.VMEM((BLK,), jnp.int32),
            pltpu.VMEM((BLK, DIM), jnp.float32),
            pltpu.VMEM((BLK, DIM), jnp.float32),
            pltpu.VMEM((BLK, DIM), jnp.float32),
            pltpu.VMEM((BLK,), jnp.float32),
            pltpu.VMEM((BLK,), jnp.float32),
            pltpu.SemaphoreType.DMA,
        ],
    )


def kernel(clause_entity_embedding, relation_embedding,
           concl_heads, concl_tails, concl_rel,
           premise_heads, premise_tails, premise_rel,
           conj_premise_heads1, conj_premise_tails1, conj_premise_rel1,
           conj_premise_heads2, conj_premise_tails2, conj_premise_rel2):
    concl, prem, conj = _build()(
        clause_entity_embedding, relation_embedding,
        concl_heads, concl_tails, concl_rel,
        premise_heads, premise_tails, premise_rel,
        conj_premise_heads1, conj_premise_tails1, conj_premise_rel1,
        conj_premise_heads2, conj_premise_tails2, conj_premise_rel2)
    return (concl, prem, conj)
